# Initial kernel scaffold; baseline (speedup 1.0000x reference)
import jax
import jax.numpy as jnp
from jax.experimental import pallas as pl


def kernel(user_id, song_id, hist_ids, hist_weights, user_table, item_table, W1, b1, W2, b2, W3, b3, W_out, b_out):
    raise NotImplementedError("write your pallas kernel here")


# speedup vs baseline: 3.5735x; 3.5735x over previous
import functools

import jax
import jax.numpy as jnp
from jax import lax
from jax.experimental import pallas as pl
from jax.experimental.pallas import tpu as pltpu
from jax.experimental.pallas import tpu_sc as plsc

_NC = 2
_NS = 16
_NW = _NC * _NS


def _sc_gather(user_table, item_table, user_id, song_id, hist_flat):
    B = user_id.shape[0]
    NH = hist_flat.shape[0]
    E = user_table.shape[1]
    b_per_w = B // _NW
    h_per_w = NH // _NW
    CH = 1600
    n_chunks = h_per_w // CH

    mesh = plsc.VectorSubcoreMesh(core_axis_name="c", subcore_axis_name="s")

    @functools.partial(
        pl.kernel,
        mesh=mesh,
        compiler_params=pltpu.CompilerParams(use_tc_tiling_on_sc=False),
        out_type=[
            jax.ShapeDtypeStruct((B, E), jnp.float32),
            jax.ShapeDtypeStruct((B, E), jnp.float32),
            jax.ShapeDtypeStruct((NH, E), jnp.float32),
        ],
        scratch_types=[
            pltpu.VMEM((b_per_w,), jnp.int32),
            pltpu.VMEM((b_per_w, E), jnp.float32),
            pltpu.VMEM((CH,), jnp.int32),
            pltpu.VMEM((CH, E), jnp.float32),
            pltpu.SemaphoreType.DMA,
        ],
    )
    def gather_kernel(ut_hbm, it_hbm, uid_hbm, sid_hbm, hid_hbm,
                      u_out, v_out, h_out,
                      idx_b, rows_b, idx_h, rows_h, sem):
        wid = lax.axis_index("s") * _NC + lax.axis_index("c")
        base = wid * b_per_w
        pltpu.sync_copy(uid_hbm.at[pl.ds(base, b_per_w)], idx_b)
        pltpu.async_copy(ut_hbm.at[idx_b], rows_b, sem).wait()
        pltpu.sync_copy(rows_b, u_out.at[pl.ds(base, b_per_w)])
        pltpu.sync_copy(sid_hbm.at[pl.ds(base, b_per_w)], idx_b)
        pltpu.async_copy(it_hbm.at[idx_b], rows_b, sem).wait()
        pltpu.sync_copy(rows_b, v_out.at[pl.ds(base, b_per_w)])
        hbase = wid * h_per_w

        @pl.loop(0, n_chunks)
        def _(c):
            off = hbase + c * CH
            pltpu.sync_copy(hid_hbm.at[pl.ds(off, CH)], idx_h)
            pltpu.async_copy(it_hbm.at[idx_h], rows_h, sem).wait()
            pltpu.sync_copy(rows_h, h_out.at[pl.ds(off, CH)])

    return gather_kernel(user_table, item_table, user_id, song_id, hist_flat)


def _tc_mlp(u, v, h3, hist_weights, W1, b1, W2, b2, W3, b3, W_out, b_out):
    B, H, E = h3.shape
    BS = 512
    grid = (B // BS,)

    def body(u_ref, v_ref, h_ref, w_ref,
             W1_ref, b1_ref, W2_ref, b2_ref, W3_ref, b3_ref,
             Wo_ref, bo_ref, out_ref):
        w = w_ref[...]
        wn = w / (jnp.sum(w, axis=1, keepdims=True) + 1e-8)
        h = h_ref[...]
        hist = jnp.sum(h * wn[:, :, None], axis=1)
        uu = u_ref[...]
        vv = v_ref[...]
        x = jnp.concatenate([uu, vv, hist], axis=1)
        x = jnp.maximum(jnp.dot(x, W1_ref[...],
                                preferred_element_type=jnp.float32)
                        + b1_ref[...][None, :], 0.0)
        x = jnp.maximum(jnp.dot(x, W2_ref[...],
                                preferred_element_type=jnp.float32)
                        + b2_ref[...][None, :], 0.0)
        x = jnp.maximum(jnp.dot(x, W3_ref[...],
                                preferred_element_type=jnp.float32)
                        + b3_ref[...][None, :], 0.0)
        y = jnp.concatenate([uu * vv, x], axis=1)
        out_ref[...] = (jnp.dot(y, Wo_ref[...],
                                preferred_element_type=jnp.float32)
                        + bo_ref[...][None, :])

    rep = lambda *shape: pl.BlockSpec(shape, lambda i: (0,) * len(shape))
    return pl.pallas_call(
        body,
        grid=grid,
        in_specs=[
            pl.BlockSpec((BS, E), lambda i: (i, 0)),
            pl.BlockSpec((BS, E), lambda i: (i, 0)),
            pl.BlockSpec((BS, H, E), lambda i: (i, 0, 0)),
            pl.BlockSpec((BS, H), lambda i: (i, 0)),
            rep(*W1.shape), rep(*b1.shape),
            rep(*W2.shape), rep(*b2.shape),
            rep(*W3.shape), rep(*b3.shape),
            rep(*W_out.shape), rep(*b_out.shape),
        ],
        out_specs=pl.BlockSpec((BS, 1), lambda i: (i, 0)),
        out_shape=jax.ShapeDtypeStruct((B, 1), jnp.float32),
    )(u, v, h3, hist_weights, W1, b1, W2, b2, W3, b3, W_out, b_out)


def kernel(user_id, song_id, hist_ids, hist_weights, user_table, item_table,
           W1, b1, W2, b2, W3, b3, W_out, b_out):
    B, H = hist_ids.shape
    E = user_table.shape[1]
    hist_flat = hist_ids.reshape(-1)
    u, v, h = _sc_gather(user_table, item_table, user_id, song_id, hist_flat)
    h3 = h.reshape(B, H, E)
    return _tc_mlp(u, v, h3, hist_weights,
                   W1, b1, W2, b2, W3, b3, W_out, b_out)


# speedup vs baseline: 5.6924x; 1.5930x over previous
import functools

import jax
import jax.numpy as jnp
from jax import lax
from jax.experimental import pallas as pl
from jax.experimental.pallas import tpu as pltpu
from jax.experimental.pallas import tpu_sc as plsc

_NC = 2
_NS = 16
_NW = _NC * _NS
_L = 16


def _sc_gather_pool(user_table, item_table, user_id, song_id,
                    hist_flat, hw_flat):
    B = user_id.shape[0]
    NH = hist_flat.shape[0]
    E = user_table.shape[1]
    H = NH // B
    b_per_w = B // _NW
    CB = 8
    CH = CB * H
    n_chunks = b_per_w // CB

    mesh = plsc.VectorSubcoreMesh(core_axis_name="c", subcore_axis_name="s")

    @functools.partial(
        pl.kernel,
        mesh=mesh,
        compiler_params=pltpu.CompilerParams(use_tc_tiling_on_sc=False,
                                             needs_layout_passes=False),
        out_type=[
            jax.ShapeDtypeStruct((B, E), jnp.float32),
            jax.ShapeDtypeStruct((B, E), jnp.float32),
            jax.ShapeDtypeStruct((B * E,), jnp.float32),
        ],
        scratch_types=[
            pltpu.VMEM((b_per_w,), jnp.int32),
            pltpu.VMEM((b_per_w, E), jnp.float32),
            pltpu.VMEM((CH,), jnp.int32),
            pltpu.VMEM((CH,), jnp.int32),
            pltpu.VMEM((CH, E), jnp.float32),
            pltpu.VMEM((CH, E), jnp.float32),
            pltpu.VMEM((b_per_w * H,), jnp.float32),
            pltpu.VMEM((b_per_w * E,), jnp.float32),
            pltpu.SemaphoreType.DMA,
            pltpu.SemaphoreType.DMA,
            pltpu.SemaphoreType.DMA,
        ],
    )
    def gather_kernel(ut_hbm, it_hbm, uid_hbm, sid_hbm, hid_hbm, hw_hbm,
                      u_out, v_out, p_out,
                      idx_b, rows_b, idx_h0, idx_h1, rows_h0, rows_h1,
                      wv, pool_buf, sem_u, sem0, sem1):
        wid = lax.axis_index("s") * _NC + lax.axis_index("c")
        base = wid * b_per_w
        hbase = base * H

        wcopy = pltpu.make_async_copy(
            hw_hbm.at[pl.ds(hbase, b_per_w * H)], wv, sem1)
        wcopy.start()

        pltpu.sync_copy(uid_hbm.at[pl.ds(base, b_per_w)], idx_b)
        pltpu.async_copy(ut_hbm.at[idx_b], rows_b, sem_u).wait()
        pltpu.sync_copy(rows_b, u_out.at[pl.ds(base, b_per_w)])
        pltpu.sync_copy(sid_hbm.at[pl.ds(base, b_per_w)], idx_b)
        pltpu.async_copy(it_hbm.at[idx_b], rows_b, sem_u).wait()
        pltpu.sync_copy(rows_b, v_out.at[pl.ds(base, b_per_w)])
        wcopy.wait()

        col = [lax.iota(jnp.int32, _L) + k * _L for k in range(E // _L)]

        def start_gather(c, idx_h, rows_h, sem):
            off = hbase + c * CH
            pltpu.sync_copy(hid_hbm.at[pl.ds(off, CH)], idx_h)
            pltpu.make_async_copy(it_hbm.at[idx_h], rows_h, sem).start()

        def compute_chunk(c, idx_h, rows_h, sem):
            pltpu.make_async_copy(it_hbm.at[idx_h], rows_h, sem).wait()

            @pl.loop(0, CB)
            def _(b):
                rbase = b * H
                wbase = c * CH + rbase

                def jstep(j, acc):
                    rvec = jnp.full((_L,), rbase + j, dtype=jnp.int32)
                    wvec = plsc.load_gather(
                        wv, [jnp.full((_L,), wbase + j, dtype=jnp.int32)])
                    return tuple(
                        acc[k] + wvec * plsc.load_gather(rows_h, [rvec, col[k]])
                        for k in range(E // _L))

                acc = lax.fori_loop(
                    0, H, jstep,
                    tuple(jnp.zeros((_L,), jnp.float32)
                          for _ in range(E // _L)))
                pbase = (c * CB + b) * E
                for k in range(E // _L):
                    pool_buf[pl.ds(pbase + k * _L, _L)] = acc[k]

        start_gather(0, idx_h0, rows_h0, sem0)

        @pl.loop(0, n_chunks // 2)
        def _(cc):
            c = cc * 2

            start_gather(c + 1, idx_h1, rows_h1, sem1)
            compute_chunk(c, idx_h0, rows_h0, sem0)

            @pl.when(c + 2 < n_chunks)
            def _():
                start_gather(c + 2, idx_h0, rows_h0, sem0)
            compute_chunk(c + 1, idx_h1, rows_h1, sem1)

        pltpu.sync_copy(pool_buf, p_out.at[pl.ds(base * E, b_per_w * E)])

    return gather_kernel(user_table, item_table, user_id, song_id,
                         hist_flat, hw_flat)


def _tc_mlp(u, v, pooled, hist_weights, W1, b1, W2, b2, W3, b3, W_out, b_out):
    B, E = u.shape
    H = hist_weights.shape[1]
    BS = 512
    grid = (B // BS,)

    def body(u_ref, v_ref, p_ref, w_ref,
             W1_ref, b1_ref, W2_ref, b2_ref, W3_ref, b3_ref,
             Wo_ref, bo_ref, out_ref):
        w = w_ref[...]
        wsum = jnp.sum(w, axis=1, keepdims=True) + 1e-8
        hist = p_ref[...] / wsum
        uu = u_ref[...]
        vv = v_ref[...]
        x = jnp.concatenate([uu, vv, hist], axis=1)
        x = jnp.maximum(jnp.dot(x, W1_ref[...],
                                preferred_element_type=jnp.float32)
                        + b1_ref[...][None, :], 0.0)
        x = jnp.maximum(jnp.dot(x, W2_ref[...],
                                preferred_element_type=jnp.float32)
                        + b2_ref[...][None, :], 0.0)
        x = jnp.maximum(jnp.dot(x, W3_ref[...],
                                preferred_element_type=jnp.float32)
                        + b3_ref[...][None, :], 0.0)
        y = jnp.concatenate([uu * vv, x], axis=1)
        out_ref[...] = (jnp.dot(y, Wo_ref[...],
                                preferred_element_type=jnp.float32)
                        + bo_ref[...][None, :])

    rep = lambda *shape: pl.BlockSpec(shape, lambda i: (0,) * len(shape))
    return pl.pallas_call(
        body,
        grid=grid,
        in_specs=[
            pl.BlockSpec((BS, E), lambda i: (i, 0)),
            pl.BlockSpec((BS, E), lambda i: (i, 0)),
            pl.BlockSpec((BS, E), lambda i: (i, 0)),
            pl.BlockSpec((BS, H), lambda i: (i, 0)),
            rep(*W1.shape), rep(*b1.shape),
            rep(*W2.shape), rep(*b2.shape),
            rep(*W3.shape), rep(*b3.shape),
            rep(*W_out.shape), rep(*b_out.shape),
        ],
        out_specs=pl.BlockSpec((BS, 1), lambda i: (i, 0)),
        out_shape=jax.ShapeDtypeStruct((B, 1), jnp.float32),
    )(u, v, pooled, hist_weights, W1, b1, W2, b2, W3, b3, W_out, b_out)


def kernel(user_id, song_id, hist_ids, hist_weights, user_table, item_table,
           W1, b1, W2, b2, W3, b3, W_out, b_out):
    B, H = hist_ids.shape
    E = user_table.shape[1]
    hist_flat = hist_ids.reshape(-1)
    hw_flat = hist_weights.reshape(-1)
    u, v, pooled_flat = _sc_gather_pool(user_table, item_table,
                                       user_id, song_id, hist_flat, hw_flat)
    pooled = pooled_flat.reshape(B, E)
    return _tc_mlp(u, v, pooled, hist_weights,
                   W1, b1, W2, b2, W3, b3, W_out, b_out)
